# Initial kernel scaffold; baseline (speedup 1.0000x reference)
import jax
import jax.numpy as jnp
from jax.experimental import pallas as pl


def kernel(gt_pc, predict_pc, neighbor_id_lstlst, neighbor_num_lst):
    raise NotImplementedError("write your pallas kernel here")


# speedup vs baseline: 8.3433x; 8.3433x over previous
import functools

import jax
import jax.numpy as jnp
from jax import lax
from jax.experimental import pallas as pl
from jax.experimental.pallas import tpu as pltpu
from jax.experimental.pallas import tpu_sc as plsc

BATCH = 32
POINT_NUM = 6890
MAX_NB = 8
LANES = 16
PADDED = 6896
GROUPS = PADDED // LANES
NB_SLOTS = MAX_NB - 1
INV_N = 1.0 / (BATCH * POINT_NUM * 3)


def _sc_body(gt_hbm, pr_hbm, nbt_hbm, nn_hbm, out_hbm,
             d0, d1, d2, t0, t1, t2, nbt_v, nn_v, o_v):
    wid = lax.axis_index("s") * 2 + lax.axis_index("c")
    b = wid

    pltpu.sync_copy(gt_hbm.at[pl.ds((b * 3 + 0) * PADDED, PADDED)], d0)
    pltpu.sync_copy(gt_hbm.at[pl.ds((b * 3 + 1) * PADDED, PADDED)], d1)
    pltpu.sync_copy(gt_hbm.at[pl.ds((b * 3 + 2) * PADDED, PADDED)], d2)
    pltpu.sync_copy(pr_hbm.at[pl.ds((b * 3 + 0) * PADDED, PADDED)], t0)
    pltpu.sync_copy(pr_hbm.at[pl.ds((b * 3 + 1) * PADDED, PADDED)], t1)
    pltpu.sync_copy(pr_hbm.at[pl.ds((b * 3 + 2) * PADDED, PADDED)], t2)
    pltpu.sync_copy(nbt_hbm, nbt_v)
    pltpu.sync_copy(nn_hbm, nn_v)

    def sub_body(g, carry):
        s = pl.ds(g * LANES, LANES)
        d0[s] = d0[s] - t0[s]
        d1[s] = d1[s] - t1[s]
        d2[s] = d2[s] - t2[s]
        return carry

    lax.fori_loop(0, GROUPS, sub_body, 0, unroll=2)

    def main_body(g, carry):
        lap, geo = carry
        s = pl.ds(g * LANES, LANES)
        nnv = nn_v[s]
        x = d0[s]
        y = d1[s]
        z = d2[s]
        ax = x * nnv
        ay = y * nnv
        az = z * nnv
        geo = geo + jnp.abs(x) + jnp.abs(y) + jnp.abs(z)
        for n in range(NB_SLOTS):
            idx = nbt_v[n, s]
            ax = ax - plsc.load_gather(d0, [idx])
            ay = ay - plsc.load_gather(d1, [idx])
            az = az - plsc.load_gather(d2, [idx])
        lap = lap + jnp.abs(ax) + jnp.abs(ay) + jnp.abs(az)
        return lap, geo

    zero = jnp.zeros((LANES,), jnp.float32)
    lap, geo = lax.fori_loop(0, GROUPS, main_body, (zero, zero))
    o_v[...] = (lap + geo) * INV_N
    pltpu.sync_copy(o_v, out_hbm.at[pl.ds(b * LANES, LANES)])


@jax.jit
def _mcloss(gt_t, pr_t, nbt, nn_p):
    call = pl.kernel(
        _sc_body,
        out_type=jax.ShapeDtypeStruct((BATCH * LANES,), jnp.float32),
        mesh=plsc.VectorSubcoreMesh(
            core_axis_name="c", subcore_axis_name="s",
            num_cores=2, num_subcores=16),
        compiler_params=pltpu.CompilerParams(needs_layout_passes=False),
        scratch_types=[
            pltpu.VMEM((PADDED,), jnp.float32),
            pltpu.VMEM((PADDED,), jnp.float32),
            pltpu.VMEM((PADDED,), jnp.float32),
            pltpu.VMEM((PADDED,), jnp.float32),
            pltpu.VMEM((PADDED,), jnp.float32),
            pltpu.VMEM((PADDED,), jnp.float32),
            pltpu.VMEM((NB_SLOTS, PADDED), jnp.int32),
            pltpu.VMEM((PADDED,), jnp.float32),
            pltpu.VMEM((LANES,), jnp.float32),
        ],
    )
    parts = call(gt_t, pr_t, nbt, nn_p)
    return jnp.sum(parts)


def kernel(gt_pc, predict_pc, neighbor_id_lstlst, neighbor_num_lst):
    pad = PADDED - POINT_NUM
    gt_t = jnp.pad(jnp.transpose(gt_pc, (0, 2, 1)),
                   ((0, 0), (0, 0), (0, pad))).reshape(-1)
    pr_t = jnp.pad(jnp.transpose(predict_pc, (0, 2, 1)),
                   ((0, 0), (0, 0), (0, pad))).reshape(-1)
    nbt = jnp.pad(jnp.transpose(neighbor_id_lstlst[:, 1:], (1, 0)),
                  ((0, 0), (0, pad)), constant_values=POINT_NUM)
    nn_p = jnp.pad(neighbor_num_lst, (0, pad))
    return _mcloss(gt_t, pr_t, nbt, nn_p)
